# Initial kernel scaffold; baseline (speedup 1.0000x reference)
import jax
import jax.numpy as jnp
from jax.experimental import pallas as pl


def kernel(action_type, x, y, action_table, x_table, y_table):
    raise NotImplementedError("write your pallas kernel here")


# speedup vs baseline: 18.1637x; 18.1637x over previous
import functools

import jax
import jax.numpy as jnp
from jax import lax
from jax.experimental import pallas as pl
from jax.experimental.pallas import tpu as pltpu
from jax.experimental.pallas import tpu_sc as plsc

D_MODEL = 128
NUM_ACTIONS = 10
GRID_SIZE = 64
COMBINED_ROWS = NUM_ACTIONS * GRID_SIZE * GRID_SIZE

NUM_CORES = 2
NUM_SUBCORES = 16
LANES = 16
NW = NUM_CORES * NUM_SUBCORES

GATHER_ROWS = 128
GPC = 8
SCHUNK = GATHER_ROWS * GPC


def _combine_tables(action_table, x_table, y_table):
    def body(a_ref, x_ref, y_ref, o_ref):
        xr = x_ref[...]
        yr = y_ref[...]
        ar = a_ref[pl.ds(pl.program_id(0), 1), :]
        o_ref[...] = xr[:, None, :] + yr[None, :, :] + ar[0][None, None, :]

    return pl.pallas_call(
        body,
        grid=(NUM_ACTIONS,),
        in_specs=[
            pl.BlockSpec((NUM_ACTIONS, D_MODEL), lambda a: (0, 0)),
            pl.BlockSpec((GRID_SIZE, D_MODEL), lambda a: (0, 0)),
            pl.BlockSpec((GRID_SIZE, D_MODEL), lambda a: (0, 0)),
        ],
        out_specs=pl.BlockSpec(
            (GRID_SIZE, GRID_SIZE, D_MODEL), lambda a: (a, 0, 0)
        ),
        out_shape=jax.ShapeDtypeStruct(
            (NUM_ACTIONS * GRID_SIZE, GRID_SIZE, D_MODEL), jnp.float32
        ),
    )(action_table, x_table, y_table)


def _sc_lookup(n_tokens: int):
    assert n_tokens % (NW * SCHUNK) == 0
    per_w = n_tokens // NW
    n_schunks = per_w // SCHUNK
    mesh = plsc.VectorSubcoreMesh(
        core_axis_name="c", subcore_axis_name="s",
        num_cores=NUM_CORES, num_subcores=NUM_SUBCORES,
    )

    @functools.partial(
        pl.kernel,
        out_type=jax.ShapeDtypeStruct((n_tokens, D_MODEL), jnp.float32),
        mesh=mesh,
        scratch_types=[
            pltpu.VMEM((SCHUNK,), jnp.int32),
            pltpu.VMEM((SCHUNK,), jnp.int32),
            pltpu.VMEM((SCHUNK,), jnp.int32),
            pltpu.VMEM((GPC, GATHER_ROWS), jnp.int32),
            pltpu.VMEM((GATHER_ROWS, D_MODEL), jnp.float32),
            pltpu.VMEM((GATHER_ROWS, D_MODEL), jnp.float32),
            pltpu.SemaphoreType.DMA,
            pltpu.SemaphoreType.DMA,
        ],
    )
    def kern(a_hbm, x_hbm, y_hbm, axy_hbm, out_hbm,
             a_v, x_v, y_v, cidx_v, rows0, rows1, sem_g, sem_o):
        wid = lax.axis_index("s") * NUM_CORES + lax.axis_index("c")
        wbase = wid * per_w
        bufs = (rows0, rows1)

        def schunk_body(sc, carry):
            base = wbase + sc * SCHUNK
            sl = pl.ds(base, SCHUNK)
            pltpu.sync_copy(a_hbm.at[sl], a_v)
            pltpu.sync_copy(x_hbm.at[sl], x_v)
            pltpu.sync_copy(y_hbm.at[sl], y_v)

            for k in range(SCHUNK // LANES):
                g = pl.ds(k * LANES, LANES)
                av = jnp.clip(a_v[g], 0, NUM_ACTIONS - 1)
                xv = jnp.clip(x_v[g], 0, GRID_SIZE - 1)
                yv = jnp.clip(y_v[g], 0, GRID_SIZE - 1)
                cidx_v[k // GPC, pl.ds((k % GPC) * LANES, LANES)] = (
                    av * (GRID_SIZE * GRID_SIZE) + xv * GRID_SIZE + yv
                )

            gh = pltpu.async_copy(axy_hbm.at[cidx_v.at[0]], bufs[0], sem_g)
            oh = None
            for j in range(GPC):
                cur = bufs[j % 2]
                gh.wait()
                if oh is not None:
                    oh.wait()
                if j + 1 < GPC:
                    gh = pltpu.async_copy(
                        axy_hbm.at[cidx_v.at[j + 1]], bufs[(j + 1) % 2], sem_g
                    )
                oh = pltpu.async_copy(
                    cur, out_hbm.at[pl.ds(base + j * GATHER_ROWS, GATHER_ROWS)],
                    sem_o,
                )
            oh.wait()
            return carry

        lax.fori_loop(0, n_schunks, schunk_body, 0)

    return kern


def kernel(action_type, x, y, action_table, x_table, y_table):
    b, l = action_type.shape
    n = b * l
    a_flat = action_type.reshape(n).astype(jnp.int32)
    x_flat = x.reshape(n).astype(jnp.int32)
    y_flat = y.reshape(n).astype(jnp.int32)
    axy = _combine_tables(action_table, x_table, y_table).reshape(
        COMBINED_ROWS, D_MODEL
    )
    out = _sc_lookup(n)(a_flat, x_flat, y_flat, axy)
    return out.reshape(b, l, D_MODEL)


# speedup vs baseline: 25.3094x; 1.3934x over previous
import functools

import jax
import jax.numpy as jnp
from jax import lax
from jax.experimental import pallas as pl
from jax.experimental.pallas import tpu as pltpu
from jax.experimental.pallas import tpu_sc as plsc

D_MODEL = 128
NUM_ACTIONS = 10
GRID_SIZE = 64
COMBINED_ROWS = NUM_ACTIONS * GRID_SIZE * GRID_SIZE

NUM_CORES = 2
NUM_SUBCORES = 16
LANES = 16
NW = NUM_CORES * NUM_SUBCORES

GATHER_ROWS = 128
GPC = 8
SCHUNK = GATHER_ROWS * GPC


def _combine_tables(action_table, x_table, y_table):
    def body(a_ref, x_ref, y_ref, o_ref):
        xr = x_ref[...]
        yr = y_ref[...]
        ar = a_ref[pl.ds(pl.program_id(0), 1), :]
        o_ref[...] = xr[:, None, :] + yr[None, :, :] + ar[0][None, None, :]

    return pl.pallas_call(
        body,
        grid=(NUM_ACTIONS,),
        in_specs=[
            pl.BlockSpec((NUM_ACTIONS, D_MODEL), lambda a: (0, 0)),
            pl.BlockSpec((GRID_SIZE, D_MODEL), lambda a: (0, 0)),
            pl.BlockSpec((GRID_SIZE, D_MODEL), lambda a: (0, 0)),
        ],
        out_specs=pl.BlockSpec(
            (GRID_SIZE, GRID_SIZE, D_MODEL), lambda a: (a, 0, 0)
        ),
        out_shape=jax.ShapeDtypeStruct(
            (NUM_ACTIONS * GRID_SIZE, GRID_SIZE, D_MODEL), jnp.float32
        ),
    )(action_table, x_table, y_table)


NBUF = 4


def _sc_lookup(n_tokens: int):
    assert n_tokens % (NW * SCHUNK) == 0
    per_w = n_tokens // NW
    n_schunks = per_w // SCHUNK
    mesh = plsc.VectorSubcoreMesh(
        core_axis_name="c", subcore_axis_name="s",
        num_cores=NUM_CORES, num_subcores=NUM_SUBCORES,
    )

    @functools.partial(
        pl.kernel,
        out_type=jax.ShapeDtypeStruct((n_tokens, D_MODEL), jnp.float32),
        mesh=mesh,
        scratch_types=[
            pltpu.VMEM((2, SCHUNK), jnp.int32),
            pltpu.VMEM((2, SCHUNK), jnp.int32),
            pltpu.VMEM((2, SCHUNK), jnp.int32),
            pltpu.VMEM((2, GPC, GATHER_ROWS), jnp.int32),
            pltpu.VMEM((NBUF, GATHER_ROWS, D_MODEL), jnp.float32),
            pltpu.SemaphoreType.DMA,
            pltpu.SemaphoreType.DMA,
            pltpu.SemaphoreType.DMA,
        ],
    )
    def kern(a_hbm, x_hbm, y_hbm, axy_hbm, out_hbm,
             a_v, x_v, y_v, cidx_v, rows_v, sem_g, sem_o, sem_i):
        wid = lax.axis_index("s") * NUM_CORES + lax.axis_index("c")
        wbase = wid * per_w

        def start_idx(slot, base):
            sl = pl.ds(base, SCHUNK)
            pltpu.async_copy(a_hbm.at[sl], a_v.at[slot], sem_i)
            pltpu.async_copy(x_hbm.at[sl], x_v.at[slot], sem_i)
            pltpu.async_copy(y_hbm.at[sl], y_v.at[slot], sem_i)

        def drain_idx(slot):
            dummy = pl.ds(0, SCHUNK)
            pltpu.make_async_copy(a_hbm.at[dummy], a_v.at[slot], sem_i).wait()
            pltpu.make_async_copy(x_hbm.at[dummy], x_v.at[slot], sem_i).wait()
            pltpu.make_async_copy(y_hbm.at[dummy], y_v.at[slot], sem_i).wait()

        def compute_cidx(slot):
            for k in range(SCHUNK // LANES):
                g = pl.ds(k * LANES, LANES)
                av = jnp.clip(a_v[slot, g], 0, NUM_ACTIONS - 1)
                xv = jnp.clip(x_v[slot, g], 0, GRID_SIZE - 1)
                yv = jnp.clip(y_v[slot, g], 0, GRID_SIZE - 1)
                cidx_v[slot, k // GPC, pl.ds((k % GPC) * LANES, LANES)] = (
                    av * (GRID_SIZE * GRID_SIZE) + xv * GRID_SIZE + yv
                )

        def start_gather(slot, row, buf):
            pltpu.async_copy(
                axy_hbm.at[cidx_v.at[slot, row]], rows_v.at[buf], sem_g
            )

        def drain_gather(buf):
            pltpu.make_async_copy(
                axy_hbm.at[pl.ds(0, GATHER_ROWS)], rows_v.at[buf], sem_g
            ).wait()

        def start_out(buf, base):
            pltpu.async_copy(
                rows_v.at[buf], out_hbm.at[pl.ds(base, GATHER_ROWS)], sem_o
            )

        def drain_out(buf):
            pltpu.make_async_copy(
                rows_v.at[buf], out_hbm.at[pl.ds(wbase, GATHER_ROWS)], sem_o
            ).wait()

        start_idx(0, wbase)
        drain_idx(0)
        compute_cidx(0)
        start_idx(1, wbase + SCHUNK)
        start_gather(0, 0, 0)
        start_gather(0, 1, 1)

        def schunk_body(s, carry):
            p = lax.rem(s, 2)
            q = lax.rem(s + 1, 2)
            base = wbase + s * SCHUNK

            @pl.when(s + 1 < n_schunks)
            def _():
                drain_idx(q)
                compute_cidx(q)

                @pl.when(s + 2 < n_schunks)
                def _():
                    start_idx(p, base + 2 * SCHUNK)

            for j in range(GPC):
                buf = j % NBUF
                if j >= 2:
                    drain_out((j - 2) % NBUF)
                else:
                    @pl.when(s > 0)
                    def _():
                        drain_out((j - 2) % NBUF)
                drain_gather(buf)
                if j < GPC - 2:
                    start_gather(p, j + 2, (j + 2) % NBUF)
                else:
                    @pl.when(s + 1 < n_schunks)
                    def _():
                        start_gather(q, (j + 2) % GPC, (j + 2) % NBUF)
                start_out(buf, base + j * GATHER_ROWS)
            return carry

        lax.fori_loop(0, n_schunks, schunk_body, 0)
        drain_out((GPC - 2) % NBUF)
        drain_out((GPC - 1) % NBUF)

    return kern


def kernel(action_type, x, y, action_table, x_table, y_table):
    b, l = action_type.shape
    n = b * l
    a_flat = action_type.reshape(n).astype(jnp.int32)
    x_flat = x.reshape(n).astype(jnp.int32)
    y_flat = y.reshape(n).astype(jnp.int32)
    axy = _combine_tables(action_table, x_table, y_table).reshape(
        COMBINED_ROWS, D_MODEL
    )
    out = _sc_lookup(n)(a_flat, x_flat, y_flat, axy)
    return out.reshape(b, l, D_MODEL)
